# Initial kernel scaffold; baseline (speedup 1.0000x reference)
import jax
import jax.numpy as jnp
from jax.experimental import pallas as pl


def kernel(x, edge_index, W_hyb, b_hyb, W_res, b_res):
    raise NotImplementedError("write your pallas kernel here")


# speedup vs baseline: 25.1992x; 25.1992x over previous
import functools

import jax
import jax.numpy as jnp
from jax import lax
from jax.experimental import pallas as pl
from jax.experimental.pallas import tpu as pltpu
from jax.experimental.pallas import tpu_sc as plsc

NC = 2
NS = 16
CHUNK = 128


def _cdiv(a, b):
    return -(-a // b)


def _make_deg_kernel(n_nodes, nd, n_edges):
    epw = n_edges // (NC * NS)
    ch = _cdiv(epw, CHUNK)
    rows_pt = nd // NS
    mesh = plsc.VectorSubcoreMesh(core_axis_name="c", subcore_axis_name="s")

    @functools.partial(
        pl.kernel,
        out_type=jax.ShapeDtypeStruct((2, NC, nd), jnp.float32),
        mesh=mesh,
        compiler_params=pltpu.CompilerParams(use_tc_tiling_on_sc=False),
        scratch_types=[
            pltpu.VMEM((ch, CHUNK), jnp.int32),
            pltpu.VMEM((ch, CHUNK), jnp.int32),
            pltpu.VMEM((CHUNK,), jnp.float32),
            pltpu.VMEM((rows_pt,), jnp.float32),
            pltpu.VMEM_SHARED((nd,), jnp.float32),
            pltpu.VMEM_SHARED((nd,), jnp.float32),
        ],
    )
    def deg_k(src_hbm, dst_hbm, zeros_hbm, ones_hbm, out_hbm,
              sidx, didx, obuf, zbuf, acc_o, acc_i):
        c = lax.axis_index("c")
        s = lax.axis_index("s")
        wid = c * NS + s
        pltpu.sync_copy(zeros_hbm, zbuf)
        pltpu.sync_copy(zbuf, acc_o.at[pl.ds(s * rows_pt, rows_pt)])
        pltpu.sync_copy(zbuf, acc_i.at[pl.ds(s * rows_pt, rows_pt)])
        pltpu.sync_copy(ones_hbm, obuf)
        pltpu.sync_copy(src_hbm.at[wid], sidx)
        pltpu.sync_copy(dst_hbm.at[wid], didx)
        plsc.subcore_barrier()

        def chunk(j, carry):
            pltpu.sync_copy(obuf, acc_o.at[sidx.at[j]], add=True)
            pltpu.sync_copy(obuf, acc_i.at[didx.at[j]], add=True)
            return carry

        lax.fori_loop(0, ch, chunk, 0)
        plsc.subcore_barrier()
        pltpu.sync_copy(acc_o.at[pl.ds(s * rows_pt, rows_pt)], zbuf)
        pltpu.sync_copy(zbuf, out_hbm.at[0].at[c].at[pl.ds(s * rows_pt, rows_pt)])
        pltpu.sync_copy(acc_i.at[pl.ds(s * rows_pt, rows_pt)], zbuf)
        pltpu.sync_copy(zbuf, out_hbm.at[1].at[c].at[pl.ds(s * rows_pt, rows_pt)])

    return deg_k


def _make_pass_kernel(n_nodes, nd, n_edges, w):
    epc = n_edges // NS
    ch = _cdiv(epc, CHUNK)
    rows_pt = nd // NS
    mesh = plsc.VectorSubcoreMesh(core_axis_name="c", subcore_axis_name="s")

    @functools.partial(
        pl.kernel,
        out_type=jax.ShapeDtypeStruct((NC, nd, w), jnp.float32),
        mesh=mesh,
        compiler_params=pltpu.CompilerParams(use_tc_tiling_on_sc=False),
        scratch_types=[
            pltpu.VMEM((ch, CHUNK), jnp.int32),
            pltpu.VMEM((ch, CHUNK), jnp.int32),
            pltpu.VMEM((CHUNK, w), jnp.float32),
            pltpu.VMEM((rows_pt, w), jnp.float32),
            pltpu.VMEM_SHARED((nd, w), jnp.float32),
        ],
    )
    def pass_k(tbl_hbm, src_hbm, dst_hbm, zeros_hbm, out_hbm,
               sidx, didx, rows, zbuf, acc):
        c = lax.axis_index("c")
        s = lax.axis_index("s")
        pltpu.sync_copy(zeros_hbm, zbuf)
        pltpu.sync_copy(zbuf, acc.at[pl.ds(s * rows_pt, rows_pt)])
        pltpu.sync_copy(src_hbm.at[s], sidx)
        pltpu.sync_copy(dst_hbm.at[s], didx)
        plsc.subcore_barrier()

        def chunk(j, carry):
            pltpu.sync_copy(tbl_hbm.at[c].at[sidx.at[j]], rows)
            pltpu.sync_copy(rows, acc.at[didx.at[j]], add=True)
            return carry

        lax.fori_loop(0, ch, chunk, 0)
        plsc.subcore_barrier()
        pltpu.sync_copy(acc.at[pl.ds(s * rows_pt, rows_pt)], zbuf)
        pltpu.sync_copy(zbuf, out_hbm.at[c].at[pl.ds(s * rows_pt, rows_pt)])

    return pass_k


def kernel(x, edge_index, W_hyb, b_hyb, W_res, b_res):
    n = x.shape[0]
    e = edge_index.shape[1]
    d_in_dim = x.shape[1]
    hid = W_hyb.shape[2]
    nch = W_hyb.shape[0]
    hw = hid * (nch // 2)
    nd = _cdiv(n + 1, NS * 16) * NS * 16
    rows_pt = nd // NS

    src = edge_index[0].astype(jnp.int32)
    dst = edge_index[1].astype(jnp.int32)

    epw = e // (NC * NS)
    ch_d = _cdiv(epw, CHUNK)
    pad_d = jnp.full((NC * NS, ch_d * CHUNK - epw), n, jnp.int32)
    src_d = jnp.concatenate([src.reshape(NC * NS, epw), pad_d], 1).reshape(
        NC * NS, ch_d, CHUNK)
    dst_d = jnp.concatenate([dst.reshape(NC * NS, epw), pad_d], 1).reshape(
        NC * NS, ch_d, CHUNK)
    epc = e // NS
    ch_p = _cdiv(epc, CHUNK)
    pad_s = jnp.zeros((NS, ch_p * CHUNK - epc), jnp.int32)
    pad_n = jnp.full((NS, ch_p * CHUNK - epc), n, jnp.int32)
    src_p = jnp.concatenate([src.reshape(NS, epc), pad_s], 1).reshape(
        NS, ch_p, CHUNK)
    dst_p = jnp.concatenate([dst.reshape(NS, epc), pad_n], 1).reshape(
        NS, ch_p, CHUNK)

    z640_1 = jnp.zeros((rows_pt,), jnp.float32)
    ones_c = jnp.ones((CHUNK,), jnp.float32)
    zb = {wd: jnp.zeros((rows_pt, wd), jnp.float32) for wd in (hw, 16, 8)}

    Wcat = jnp.transpose(W_hyb, (1, 0, 2)).reshape(d_in_dim, nch * hid)
    bcat = b_hyb.reshape(nch * hid)

    deg = _make_deg_kernel(n, nd, e)(src_d, dst_d, z640_1, ones_c)

    def prep_body(x_ref, wc_ref, bc_ref, deg_ref, z0_ref, wv_ref, vv_ref):
        d_out = deg_ref[0, 0, :n] + deg_ref[0, 1, :n]
        d_inn = deg_ref[1, 0, :n] + deg_ref[1, 1, :n]
        u = lax.rsqrt(jnp.maximum(d_out, 1.0))
        v = lax.rsqrt(jnp.maximum(d_inn, 1.0))
        h0 = jnp.dot(x_ref[...], wc_ref[...],
                     preferred_element_type=jnp.float32) + bc_ref[...][None, :]
        z0 = h0 * u[:, None]
        z0_ref[0] = z0[:, :hw]
        z0_ref[1] = z0[:, hw:]
        wv_ref[...] = (u * v)[:, None]
        vv_ref[...] = v[:, None]

    z0, wv, vv = pl.pallas_call(
        prep_body,
        out_shape=[jax.ShapeDtypeStruct((NC, n, hw), jnp.float32),
                   jax.ShapeDtypeStruct((n, 1), jnp.float32),
                   jax.ShapeDtypeStruct((n, 1), jnp.float32)],
    )(x, Wcat, bcat, deg)

    pass24 = _make_pass_kernel(n, nd, e, hw)
    pass16 = _make_pass_kernel(n, nd, e, 16)
    pass8 = _make_pass_kernel(n, nd, e, 8)

    y1 = pass24(z0, src_p, dst_p, zb[hw])

    def upd1_body(z_ref, y_ref, w_ref, s_ref, a_ref):
        w2 = w_ref[...]
        sn = z_ref[0] - w2 * y_ref[0, :n, :]
        sp = w2 * y_ref[1, :n, :]
        s_ref[0] = sn
        s_ref[1] = sp
        a_ref[0] = sn[:, 8:]
        a_ref[1] = sp[:, 8:]

    s1, a1 = pl.pallas_call(
        upd1_body,
        out_shape=[jax.ShapeDtypeStruct((NC, n, hw), jnp.float32),
                   jax.ShapeDtypeStruct((NC, n, 16), jnp.float32)],
    )(z0, y1, wv)

    y2 = pass16(a1, src_p, dst_p, zb[16])

    def upd2_body(s_in, y_ref, w_ref, s_ref, a_ref):
        w2 = w_ref[...]
        sn = s_in[0, :, 8:] - w2 * y_ref[0, :n, :]
        sp = w2 * y_ref[1, :n, :]
        s_ref[0] = jnp.concatenate([s_in[0, :, :8], sn], axis=1)
        s_ref[1] = jnp.concatenate([s_in[1, :, :8], sp], axis=1)
        a_ref[0] = sn[:, 8:]
        a_ref[1] = sp[:, 8:]

    s2, a2 = pl.pallas_call(
        upd2_body,
        out_shape=[jax.ShapeDtypeStruct((NC, n, hw), jnp.float32),
                   jax.ShapeDtypeStruct((NC, n, 8), jnp.float32)],
    )(s1, y2, wv)

    y3 = pass8(a2, src_p, dst_p, zb[8])

    def upd3_body(s_in, y_ref, w_ref, f_ref):
        w2 = w_ref[...]
        sn = jnp.concatenate(
            [s_in[0, :, :16], s_in[0, :, 16:] - w2 * y_ref[0, :n, :]], axis=1)
        sp = jnp.concatenate([s_in[1, :, :16], w2 * y_ref[1, :n, :]], axis=1)
        f_ref[0] = jnp.maximum(sn, 0.0)
        f_ref[1] = jnp.maximum(sp, 0.0)

    f = pl.pallas_call(
        upd3_body,
        out_shape=jax.ShapeDtypeStruct((NC, n, hw), jnp.float32),
    )(s2, y3, wv)

    y4 = pass24(f, src_p, dst_p, zb[hw])

    def fin_body(y_ref, v_ref, wr_ref, br_ref, o_ref):
        h = jnp.concatenate([y_ref[0, :n, :], y_ref[1, :n, :]], axis=1)
        h = h * v_ref[...]
        o_ref[...] = jnp.dot(h, wr_ref[...],
                             preferred_element_type=jnp.float32) + br_ref[...][None, :]

    out = pl.pallas_call(
        fin_body,
        out_shape=jax.ShapeDtypeStruct((n, W_res.shape[1]), jnp.float32),
    )(y4, vv, W_res, b_res)
    return out


# speedup vs baseline: 44.1025x; 1.7502x over previous
import functools

import jax
import jax.numpy as jnp
from jax import lax
from jax.experimental import pallas as pl
from jax.experimental.pallas import tpu as pltpu
from jax.experimental.pallas import tpu_sc as plsc

NC = 2
NS = 16
CHUNK = 128


def _cdiv(a, b):
    return -(-a // b)


def _make_deg_kernel(n_nodes, nd, n_edges):
    epw = n_edges // (NC * NS)
    ch = _cdiv(epw, CHUNK)
    rows_pt = nd // NS
    mesh = plsc.VectorSubcoreMesh(core_axis_name="c", subcore_axis_name="s")

    @functools.partial(
        pl.kernel,
        out_type=jax.ShapeDtypeStruct((2, NC, nd), jnp.float32),
        mesh=mesh,
        compiler_params=pltpu.CompilerParams(use_tc_tiling_on_sc=False),
        scratch_types=[
            pltpu.VMEM((ch, CHUNK), jnp.int32),
            pltpu.VMEM((ch, CHUNK), jnp.int32),
            pltpu.VMEM((CHUNK,), jnp.float32),
            pltpu.VMEM((rows_pt,), jnp.float32),
            pltpu.VMEM_SHARED((nd,), jnp.float32),
            pltpu.VMEM_SHARED((nd,), jnp.float32),
        ],
    )
    def deg_k(src_hbm, dst_hbm, zeros_hbm, ones_hbm, out_hbm,
              sidx, didx, obuf, zbuf, acc_o, acc_i):
        c = lax.axis_index("c")
        s = lax.axis_index("s")
        wid = c * NS + s
        pltpu.sync_copy(zeros_hbm, zbuf)
        pltpu.sync_copy(zbuf, acc_o.at[pl.ds(s * rows_pt, rows_pt)])
        pltpu.sync_copy(zbuf, acc_i.at[pl.ds(s * rows_pt, rows_pt)])
        pltpu.sync_copy(ones_hbm, obuf)
        pltpu.sync_copy(src_hbm.at[wid], sidx)
        pltpu.sync_copy(dst_hbm.at[wid], didx)
        plsc.subcore_barrier()

        def chunk(j, carry):
            pltpu.sync_copy(obuf, acc_o.at[sidx.at[j]], add=True)
            pltpu.sync_copy(obuf, acc_i.at[didx.at[j]], add=True)
            return carry

        lax.fori_loop(0, ch, chunk, 0)
        plsc.subcore_barrier()
        pltpu.sync_copy(acc_o.at[pl.ds(s * rows_pt, rows_pt)], zbuf)
        pltpu.sync_copy(zbuf, out_hbm.at[0].at[c].at[pl.ds(s * rows_pt, rows_pt)])
        pltpu.sync_copy(acc_i.at[pl.ds(s * rows_pt, rows_pt)], zbuf)
        pltpu.sync_copy(zbuf, out_hbm.at[1].at[c].at[pl.ds(s * rows_pt, rows_pt)])

    return deg_k


def _make_pass_kernel(n_nodes, nd, n_edges, w):
    epc = n_edges // NS
    ch = _cdiv(epc, CHUNK)
    rows_pt = nd // NS
    nbuf = 6
    sdep = 2
    mesh = plsc.VectorSubcoreMesh(core_axis_name="c", subcore_axis_name="s")

    @functools.partial(
        pl.kernel,
        out_type=jax.ShapeDtypeStruct((NC, nd, w), jnp.float32),
        mesh=mesh,
        compiler_params=pltpu.CompilerParams(use_tc_tiling_on_sc=False),
        scratch_types=[
            pltpu.VMEM((ch, CHUNK), jnp.int32),
            pltpu.VMEM((ch, CHUNK), jnp.int32),
            pltpu.VMEM((nbuf, CHUNK, w), jnp.float32),
            pltpu.VMEM((rows_pt, w), jnp.float32),
            pltpu.VMEM_SHARED((nd, w), jnp.float32),
            pltpu.SemaphoreType.DMA,
            pltpu.SemaphoreType.DMA,
        ],
    )
    def pass_k(tbl_hbm, src_hbm, dst_hbm, zeros_hbm, out_hbm,
               sidx, didx, rows, zbuf, acc, gsem, ssem):
        c = lax.axis_index("c")
        s = lax.axis_index("s")
        pltpu.sync_copy(zeros_hbm, zbuf)
        pltpu.sync_copy(zbuf, acc.at[pl.ds(s * rows_pt, rows_pt)])
        pltpu.sync_copy(src_hbm.at[s], sidx)
        pltpu.sync_copy(dst_hbm.at[s], didx)
        plsc.subcore_barrier()

        for b in range(nbuf):
            pltpu.async_copy(tbl_hbm.at[c].at[sidx.at[b]], rows.at[b], gsem)

        def chunk(j, carry):
            b = lax.rem(j, nbuf)
            pltpu.make_async_copy(
                tbl_hbm.at[c].at[sidx.at[j]], rows.at[b], gsem).wait()
            pltpu.async_copy(rows.at[b], acc.at[didx.at[j]], ssem, add=True)

            @pl.when(j >= sdep)
            def _():
                jn = j - sdep + nbuf
                bn = lax.rem(jn, nbuf)
                pltpu.make_async_copy(
                    rows.at[bn], acc.at[didx.at[j]], ssem).wait()

                @pl.when(jn < ch)
                def _():
                    pltpu.async_copy(
                        tbl_hbm.at[c].at[sidx.at[jn]], rows.at[bn], gsem)

            return carry

        lax.fori_loop(0, ch, chunk, 0)
        for _ in range(sdep):
            pltpu.make_async_copy(rows.at[0], acc.at[didx.at[0]], ssem).wait()
        plsc.subcore_barrier()
        pltpu.sync_copy(acc.at[pl.ds(s * rows_pt, rows_pt)], zbuf)
        pltpu.sync_copy(zbuf, out_hbm.at[c].at[pl.ds(s * rows_pt, rows_pt)])

    return pass_k


def kernel(x, edge_index, W_hyb, b_hyb, W_res, b_res):
    n = x.shape[0]
    e = edge_index.shape[1]
    d_in_dim = x.shape[1]
    hid = W_hyb.shape[2]
    nch = W_hyb.shape[0]
    hw = hid * (nch // 2)
    nd = _cdiv(n + 1, NS * 16) * NS * 16
    rows_pt = nd // NS

    src = edge_index[0].astype(jnp.int32)
    dst = edge_index[1].astype(jnp.int32)

    epw = e // (NC * NS)
    ch_d = _cdiv(epw, CHUNK)
    pad_d = jnp.full((NC * NS, ch_d * CHUNK - epw), n, jnp.int32)
    src_d = jnp.concatenate([src.reshape(NC * NS, epw), pad_d], 1).reshape(
        NC * NS, ch_d, CHUNK)
    dst_d = jnp.concatenate([dst.reshape(NC * NS, epw), pad_d], 1).reshape(
        NC * NS, ch_d, CHUNK)
    epc = e // NS
    ch_p = _cdiv(epc, CHUNK)
    pad_s = jnp.zeros((NS, ch_p * CHUNK - epc), jnp.int32)
    pad_n = jnp.full((NS, ch_p * CHUNK - epc), n, jnp.int32)
    src_p = jnp.concatenate([src.reshape(NS, epc), pad_s], 1).reshape(
        NS, ch_p, CHUNK)
    dst_p = jnp.concatenate([dst.reshape(NS, epc), pad_n], 1).reshape(
        NS, ch_p, CHUNK)

    z640_1 = jnp.zeros((rows_pt,), jnp.float32)
    ones_c = jnp.ones((CHUNK,), jnp.float32)
    zb = {wd: jnp.zeros((rows_pt, wd), jnp.float32) for wd in (hw, 16, 8)}

    Wcat = jnp.transpose(W_hyb, (1, 0, 2)).reshape(d_in_dim, nch * hid)
    bcat = b_hyb.reshape(nch * hid)

    deg = _make_deg_kernel(n, nd, e)(src_d, dst_d, z640_1, ones_c)

    def prep_body(x_ref, wc_ref, bc_ref, deg_ref, z0_ref, wv_ref, vv_ref):
        d_out = deg_ref[0, 0, :n] + deg_ref[0, 1, :n]
        d_inn = deg_ref[1, 0, :n] + deg_ref[1, 1, :n]
        u = lax.rsqrt(jnp.maximum(d_out, 1.0))
        v = lax.rsqrt(jnp.maximum(d_inn, 1.0))
        h0 = jnp.dot(x_ref[...], wc_ref[...],
                     preferred_element_type=jnp.float32) + bc_ref[...][None, :]
        z0 = h0 * u[:, None]
        z0_ref[0] = z0[:, :hw]
        z0_ref[1] = z0[:, hw:]
        wv_ref[...] = (u * v)[:, None]
        vv_ref[...] = v[:, None]

    z0, wv, vv = pl.pallas_call(
        prep_body,
        out_shape=[jax.ShapeDtypeStruct((NC, n, hw), jnp.float32),
                   jax.ShapeDtypeStruct((n, 1), jnp.float32),
                   jax.ShapeDtypeStruct((n, 1), jnp.float32)],
    )(x, Wcat, bcat, deg)

    pass24 = _make_pass_kernel(n, nd, e, hw)
    pass16 = _make_pass_kernel(n, nd, e, 16)
    pass8 = _make_pass_kernel(n, nd, e, 8)

    y1 = pass24(z0, src_p, dst_p, zb[hw])

    def upd1_body(z_ref, y_ref, w_ref, s_ref, a_ref):
        w2 = w_ref[...]
        sn = z_ref[0] - w2 * y_ref[0, :n, :]
        sp = w2 * y_ref[1, :n, :]
        s_ref[0] = sn
        s_ref[1] = sp
        a_ref[0] = sn[:, 8:]
        a_ref[1] = sp[:, 8:]

    s1, a1 = pl.pallas_call(
        upd1_body,
        out_shape=[jax.ShapeDtypeStruct((NC, n, hw), jnp.float32),
                   jax.ShapeDtypeStruct((NC, n, 16), jnp.float32)],
    )(z0, y1, wv)

    y2 = pass16(a1, src_p, dst_p, zb[16])

    def upd2_body(s_in, y_ref, w_ref, s_ref, a_ref):
        w2 = w_ref[...]
        sn = s_in[0, :, 8:] - w2 * y_ref[0, :n, :]
        sp = w2 * y_ref[1, :n, :]
        s_ref[0] = jnp.concatenate([s_in[0, :, :8], sn], axis=1)
        s_ref[1] = jnp.concatenate([s_in[1, :, :8], sp], axis=1)
        a_ref[0] = sn[:, 8:]
        a_ref[1] = sp[:, 8:]

    s2, a2 = pl.pallas_call(
        upd2_body,
        out_shape=[jax.ShapeDtypeStruct((NC, n, hw), jnp.float32),
                   jax.ShapeDtypeStruct((NC, n, 8), jnp.float32)],
    )(s1, y2, wv)

    y3 = pass8(a2, src_p, dst_p, zb[8])

    def upd3_body(s_in, y_ref, w_ref, f_ref):
        w2 = w_ref[...]
        sn = jnp.concatenate(
            [s_in[0, :, :16], s_in[0, :, 16:] - w2 * y_ref[0, :n, :]], axis=1)
        sp = jnp.concatenate([s_in[1, :, :16], w2 * y_ref[1, :n, :]], axis=1)
        f_ref[0] = jnp.maximum(sn, 0.0)
        f_ref[1] = jnp.maximum(sp, 0.0)

    f = pl.pallas_call(
        upd3_body,
        out_shape=jax.ShapeDtypeStruct((NC, n, hw), jnp.float32),
    )(s2, y3, wv)

    y4 = pass24(f, src_p, dst_p, zb[hw])

    def fin_body(y_ref, v_ref, wr_ref, br_ref, o_ref):
        h = jnp.concatenate([y_ref[0, :n, :], y_ref[1, :n, :]], axis=1)
        h = h * v_ref[...]
        o_ref[...] = jnp.dot(h, wr_ref[...],
                             preferred_element_type=jnp.float32) + br_ref[...][None, :]

    out = pl.pallas_call(
        fin_body,
        out_shape=jax.ShapeDtypeStruct((n, W_res.shape[1]), jnp.float32),
    )(y4, vv, W_res, b_res)
    return out
